# baseline (speedup 1.0000x reference)
import functools

import jax
import jax.numpy as jnp
from jax import lax
from jax.experimental import pallas as pl
from jax.experimental.pallas import tpu as pltpu
from jax.experimental.pallas import tpu_sc as plsc

D = 1024
BN = 1024
VOCAB = 50265
NW = 32
EPW = BN // NW
LANES = 16

W_TOK = 336
CHUNK = 16
NCHUNK = W_TOK // CHUNK
CHUNK_FIELD = [0] + [1] * 2 + [2] * 18
FIRST_CHUNK = (0, 1, 3)

GCH = 32

_f32 = jnp.float32


def _splat_lane(vec, t):
    idx = jnp.full((LANES, 1), t, jnp.int32)
    dnums = lax.GatherDimensionNumbers(
        offset_dims=(), collapsed_slice_dims=(0,), start_index_map=(0,))
    return lax.gather(vec, idx, dnums, (1,),
                      mode=lax.GatherScatterMode.PROMISE_IN_BOUNDS)


def _names_fc_body(fn_ref, wtop_ref, bfc_ref, out_ref):
    out_ref[...] = (
        jnp.dot(fn_ref[...], wtop_ref[...], preferred_element_type=_f32)
        + bfc_ref[...]
    )


def _table2_body(cvec_ref, emb_ref, wbot_ref, wlin_ref, out_ref):
    h = jnp.dot(emb_ref[...], wbot_ref[...], preferred_element_type=_f32)
    h = jnp.maximum(h + cvec_ref[...], 0.0)
    out_ref[...] = jnp.dot(h, wlin_ref[...], preferred_element_type=_f32)


def _fc5_body(price_ref, rating_ref, pooled_ref, names_ref, wp_ref, wr_ref,
              wbot_ref, wlin_ref, out_ref):
    for f in range(5):
        if f == 0:
            v = jnp.dot(price_ref[...], wp_ref[...], preferred_element_type=_f32)
        elif f == 1:
            v = jnp.dot(rating_ref[...], wr_ref[...], preferred_element_type=_f32)
        else:
            v = pooled_ref[:, f - 2, :]
        h = jnp.dot(v, wbot_ref[...], preferred_element_type=_f32)
        h = jnp.maximum(h + names_ref[f, :][None, :], 0.0)
        out_ref[:, f, :] = jnp.dot(h, wlin_ref[...], preferred_element_type=_f32)
    out_ref[:, 5:8, :] = jnp.zeros_like(out_ref[:, 5:8, :])


@functools.cache
def _get_pooled_kernel():
    return functools.partial(
        pl.kernel,
        mesh=plsc.VectorSubcoreMesh(core_axis_name="c", subcore_axis_name="s"),
        out_type=jax.ShapeDtypeStruct((BN, 3, D), _f32),
        scratch_types=[
            pltpu.VMEM((EPW * W_TOK,), jnp.int32),
            pltpu.VMEM((EPW * W_TOK,), _f32),
            pltpu.VMEM((CHUNK, D), _f32),
            pltpu.VMEM((CHUNK, D), _f32),
            pltpu.VMEM((CHUNK, D), _f32),
            pltpu.VMEM((CHUNK, D), _f32),
            pltpu.VMEM((3, D), _f32),
            pltpu.SemaphoreType.DMA,
            pltpu.SemaphoreType.DMA,
            pltpu.SemaphoreType.DMA,
            pltpu.SemaphoreType.DMA,
        ],
    )(_pooled_body)


NBUF = 4


def _pooled_body(emb_hbm, idx_hbm, w_hbm, out_hbm,
                 idxs_v, ws_v, buf0, buf1, buf2, buf3, acc_v,
                 sem0, sem1, sem2, sem3):
    wid = lax.axis_index("s") * 2 + lax.axis_index("c")
    base = wid * EPW
    pltpu.sync_copy(idx_hbm.at[pl.ds(base * W_TOK, EPW * W_TOK)], idxs_v)
    pltpu.sync_copy(w_hbm.at[pl.ds(base * W_TOK, EPW * W_TOK)], ws_v)
    bufs = (buf0, buf1, buf2, buf3)
    sems = (sem0, sem1, sem2, sem3)

    def ebody(e, carry):
        ebase = e * W_TOK
        cps = [None] * NCHUNK

        def start(c):
            ivc = idxs_v[pl.ds(ebase + c * CHUNK, CHUNK)]
            cps[c] = pltpu.make_async_copy(
                emb_hbm.at[ivc], bufs[c % NBUF], sems[c % NBUF])
            cps[c].start()

        for c in range(NBUF - 1):
            start(c)
        for c in range(NCHUNK):
            if c + NBUF - 1 < NCHUNK:
                start(c + NBUF - 1)
            cps[c].wait()
            buf = bufs[c % NBUF]
            f = CHUNK_FIELD[c]
            w16 = ws_v[pl.ds(ebase + c * CHUNK, CHUNK)]
            wts = [_splat_lane(w16, t) for t in range(CHUNK)]

            overwrite = c in FIRST_CHUNK

            def ibody(i, _):
                sl = pl.ds(i * LANES, LANES)
                a = wts[0] * buf[0, sl]
                if not overwrite:
                    a = a + acc_v[f, sl]
                for t in range(1, CHUNK):
                    a = a + wts[t] * buf[t, sl]
                acc_v[f, sl] = a
                return 0

            lax.fori_loop(0, D // LANES, ibody, 0)
        pltpu.sync_copy(acc_v, out_hbm.at[base + e])
        return carry

    lax.fori_loop(0, EPW, ebody, 0)


_NBLK = 6


@functools.cache
def _get_assemble_kernel():
    return functools.partial(
        pl.kernel,
        mesh=plsc.VectorSubcoreMesh(core_axis_name="c", subcore_axis_name="s"),
        out_type=jax.ShapeDtypeStruct((BN, 133, D), _f32),
        scratch_types=[
            pltpu.VMEM((8 * _NBLK, 32), jnp.int32),
            pltpu.VMEM((8, D), _f32),
            pltpu.VMEM((5, D), _f32),
            pltpu.VMEM((32, D), _f32),
            pltpu.VMEM((32, D), _f32),
            pltpu.VMEM((32, D), _f32),
            pltpu.SemaphoreType.DMA,
            pltpu.SemaphoreType.DMA,
            pltpu.SemaphoreType.DMA,
            pltpu.SemaphoreType.DMA,
            pltpu.SemaphoreType.DMA,
            pltpu.SemaphoreType.DMA,
            pltpu.SemaphoreType.DMA,
            pltpu.SemaphoreType.DMA,
        ],
    )(_assemble_body)


def _assemble_body(table2_hbm, didx_hbm, fc8_hbm, out_hbm,
                   didx_v, buf0, tail_v, w0, w1, w2,
                   semh, sem0, sem1, sem2, semwh, semw0, semw1, semw2):
    wid = lax.axis_index("s") * 2 + lax.axis_index("c")
    base = wid * EPW

    def ebody(e, carry):
        b = base + e
        pltpu.sync_copy(didx_hbm.at[b], didx_v)

        def gather(k, dst, sem):
            cp = pltpu.make_async_copy(
                table2_hbm.at[didx_v.at[8 * k]], dst, sem)
            cp.start()
            return cp

        cp_h = pltpu.make_async_copy(fc8_hbm.at[b], buf0, semh)
        cp_h.start()
        g0 = gather(0, w2, sem2)
        g1 = gather(1, w0, sem0)
        g2 = gather(2, w1, sem1)
        g0.wait()
        cp_h.wait()

        def mv0(i, _):
            sl = pl.ds(i * LANES, LANES)
            for r in range(3):
                buf0[5 + r, sl] = w2[r, sl]
            return 0

        lax.fori_loop(0, D // LANES, mv0, 0)
        cw_h = pltpu.make_async_copy(buf0, out_hbm.at[b, pl.ds(0, 8)], semwh)
        cw_h.start()
        g3 = gather(3, w2, sem2)
        g1.wait()
        cw_a = pltpu.make_async_copy(w0, out_hbm.at[b, pl.ds(8, 32)], semw0)
        cw_a.start()
        g2.wait()
        cw_b = pltpu.make_async_copy(w1, out_hbm.at[b, pl.ds(40, 32)], semw1)
        cw_b.start()
        g3.wait()
        cw_c = pltpu.make_async_copy(w2, out_hbm.at[b, pl.ds(72, 32)], semw2)
        cw_c.start()
        cw_a.wait()
        g4 = gather(4, w0, sem0)
        g4.wait()
        cw_x = pltpu.make_async_copy(
            w0.at[pl.ds(0, 24)], out_hbm.at[b, pl.ds(104, 24)], semw0)
        cw_x.start()
        cw_b.wait()
        g5 = gather(5, w1, sem1)
        g5.wait()

        def mvt(i, _):
            sl = pl.ds(i * LANES, LANES)
            for r in range(5):
                tail_v[r, sl] = w1[r, sl]
            return 0

        lax.fori_loop(0, D // LANES, mvt, 0)
        cw_t = pltpu.make_async_copy(
            tail_v, out_hbm.at[b, pl.ds(128, 5)], semw1)
        cw_t.start()
        cw_c.wait()
        cw_x.wait()
        cw_t.wait()
        cw_h.wait()
        return carry

    lax.fori_loop(0, EPW, ebody, 0)


def kernel(field, price, rating, brand, name, category, description,
           emb_table, W_price, W_rating, W_fc, b_fc, W_lin):
    price = price.astype(_f32)
    rating = rating.astype(_f32)
    brand = brand.astype(jnp.int32)
    name = name.astype(jnp.int32)
    category = category.astype(jnp.int32)
    description = description.astype(jnp.int32)
    field = field.astype(jnp.int32)

    W_top = W_fc[:D]
    W_bot = W_fc[D:]

    brand_w = (brand != 1).astype(_f32)
    name_w = (name != 1).astype(_f32)
    cm = category != 1
    cm2 = jnp.any(cm, axis=-1)
    n2 = cm2.sum(axis=-1).astype(_f32)
    cm1 = jnp.any(cm2, axis=-1)
    n1 = cm1.sum(axis=-1).astype(_f32)
    cat_w = cm.astype(_f32) / (
        (n2[..., None, None] + 1e-6) * (n1[:, None, None, None] + 1e-6))
    zpad4 = jnp.zeros((BN, 4), _f32)
    w_all = jnp.concatenate(
        [brand_w, zpad4, name_w, cat_w.reshape(BN, 288)], axis=1)
    ipad4 = jnp.zeros((BN, 4), jnp.int32)
    idx_all = jnp.concatenate(
        [brand, ipad4, name, category.reshape(BN, 288)], axis=1)

    field_name = jnp.take(emb_table, field[:, 0], axis=0)
    fn8 = jnp.concatenate([field_name, jnp.zeros((2, D), _f32)], axis=0)

    names_fcb = pl.pallas_call(
        _names_fc_body,
        out_shape=jax.ShapeDtypeStruct((8, D), _f32),
    )(fn8, W_top, b_fc[None, :])

    tile_v = 512
    gv = pl.cdiv(VOCAB, tile_v)
    table2 = pl.pallas_call(
        _table2_body,
        grid=(gv,),
        in_specs=[
            pl.BlockSpec((1, D), lambda i: (0, 0)),
            pl.BlockSpec((tile_v, D), lambda i: (i, 0)),
            pl.BlockSpec((D, D), lambda i: (0, 0)),
            pl.BlockSpec((D, D), lambda i: (0, 0)),
        ],
        out_specs=pl.BlockSpec((tile_v, D), lambda i: (i, 0)),
        out_shape=jax.ShapeDtypeStruct((VOCAB, D), _f32),
    )(names_fcb[5:6], emb_table, W_bot, W_lin)

    pooled = _get_pooled_kernel()(
        emb_table, idx_all.reshape(-1), w_all.reshape(-1))

    bt = 256
    fc5 = pl.pallas_call(
        _fc5_body,
        grid=(BN // bt,),
        in_specs=[
            pl.BlockSpec((bt, 11), lambda i: (i, 0)),
            pl.BlockSpec((bt, 4), lambda i: (i, 0)),
            pl.BlockSpec((bt, 3, D), lambda i: (i, 0, 0)),
            pl.BlockSpec((8, D), lambda i: (0, 0)),
            pl.BlockSpec((11, D), lambda i: (0, 0)),
            pl.BlockSpec((4, D), lambda i: (0, 0)),
            pl.BlockSpec((D, D), lambda i: (0, 0)),
            pl.BlockSpec((D, D), lambda i: (0, 0)),
        ],
        out_specs=pl.BlockSpec((bt, 8, D), lambda i: (i, 0, 0)),
        out_shape=jax.ShapeDtypeStruct((BN, 8, D), _f32),
    )(price, rating, pooled, names_fcb, W_price, W_rating, W_bot, W_lin)

    zi = jnp.zeros((BN, 29), jnp.int32)
    blocks = jnp.stack(
        [jnp.concatenate([description[:, 0:3], zi[:, :29]], axis=1),
         description[:, 3:35],
         description[:, 35:67],
         description[:, 67:99],
         jnp.concatenate([description[:, 99:123], zi[:, :8]], axis=1),
         jnp.concatenate([description[:, 123:128], zi[:, :27]], axis=1)],
        axis=1)
    didx = jnp.zeros((BN, 8 * _NBLK, 32), jnp.int32)
    didx = didx.at[:, ::8, :].set(blocks)
    all_embeddings = _get_assemble_kernel()(table2, didx, fc5)

    price_mask = price.sum(axis=1, keepdims=True) != 0.0
    ones = jnp.ones((BN, 1), bool)
    all_masks = jnp.concatenate(
        [price_mask, ones, brand[:, :1] != 1, name[:, :1] != 1, ones,
         description != 1], axis=1)

    return all_embeddings, all_masks


# speedup vs baseline: 2.0682x; 2.0682x over previous
import functools

import jax
import jax.numpy as jnp
from jax import lax
from jax.experimental import pallas as pl
from jax.experimental.pallas import tpu as pltpu
from jax.experimental.pallas import tpu_sc as plsc

D = 1024
BN = 1024
VOCAB = 50265
NW = 32
EPW = BN // NW
LANES = 16

W_TOK = 336
CHUNK = 16
NCHUNK = W_TOK // CHUNK
CHUNK_FIELD = [0] + [1] * 2 + [2] * 18
FIRST_CHUNK = (0, 1, 3)

GCH = 32

_f32 = jnp.float32


def _splat_lane(vec, t):
    idx = jnp.full((LANES, 1), t, jnp.int32)
    dnums = lax.GatherDimensionNumbers(
        offset_dims=(), collapsed_slice_dims=(0,), start_index_map=(0,))
    return lax.gather(vec, idx, dnums, (1,),
                      mode=lax.GatherScatterMode.PROMISE_IN_BOUNDS)


def _names_fc_body(fn_ref, wtop_ref, bfc_ref, out_ref):
    out_ref[...] = (
        jnp.dot(fn_ref[...], wtop_ref[...], preferred_element_type=_f32)
        + bfc_ref[...]
    )


def _table2_body(cvec_ref, emb_ref, wbot_ref, wlin_ref, out_ref):
    h = jnp.dot(emb_ref[...], wbot_ref[...], preferred_element_type=_f32)
    h = jnp.maximum(h + cvec_ref[...], 0.0)
    out_ref[...] = jnp.dot(h, wlin_ref[...], preferred_element_type=_f32)


def _fc5_body(price_ref, rating_ref, pooled_ref, names_ref, wp_ref, wr_ref,
              wbot_ref, wlin_ref, out_ref):
    for f in range(5):
        if f == 0:
            v = jnp.dot(price_ref[...], wp_ref[...], preferred_element_type=_f32)
        elif f == 1:
            v = jnp.dot(rating_ref[...], wr_ref[...], preferred_element_type=_f32)
        else:
            v = pooled_ref[:, f - 2, :]
        h = jnp.dot(v, wbot_ref[...], preferred_element_type=_f32)
        h = jnp.maximum(h + names_ref[f, :][None, :], 0.0)
        out_ref[:, f, :] = jnp.dot(h, wlin_ref[...], preferred_element_type=_f32)


@functools.cache
def _get_pooled_kernel():
    return functools.partial(
        pl.kernel,
        mesh=plsc.VectorSubcoreMesh(core_axis_name="c", subcore_axis_name="s"),
        out_type=jax.ShapeDtypeStruct((BN, 3, D), _f32),
        scratch_types=[
            pltpu.VMEM((EPW * W_TOK,), jnp.int32),
            pltpu.VMEM((EPW * W_TOK,), _f32),
            pltpu.VMEM((CHUNK, D), _f32),
            pltpu.VMEM((CHUNK, D), _f32),
            pltpu.VMEM((CHUNK, D), _f32),
            pltpu.VMEM((CHUNK, D), _f32),
            pltpu.VMEM((3, D), _f32),
            pltpu.SemaphoreType.DMA,
            pltpu.SemaphoreType.DMA,
            pltpu.SemaphoreType.DMA,
            pltpu.SemaphoreType.DMA,
        ],
    )(_pooled_body)


NBUF = 4


def _pooled_body(emb_hbm, idx_hbm, w_hbm, out_hbm,
                 idxs_v, ws_v, buf0, buf1, buf2, buf3, acc_v,
                 sem0, sem1, sem2, sem3):
    wid = lax.axis_index("s") * 2 + lax.axis_index("c")
    base = wid * EPW
    pltpu.sync_copy(idx_hbm.at[pl.ds(base * W_TOK, EPW * W_TOK)], idxs_v)
    pltpu.sync_copy(w_hbm.at[pl.ds(base * W_TOK, EPW * W_TOK)], ws_v)
    bufs = (buf0, buf1, buf2, buf3)
    sems = (sem0, sem1, sem2, sem3)

    def ebody(e, carry):
        ebase = e * W_TOK
        cps = [None] * NCHUNK

        def start(c):
            ivc = idxs_v[pl.ds(ebase + c * CHUNK, CHUNK)]
            cps[c] = pltpu.make_async_copy(
                emb_hbm.at[ivc], bufs[c % NBUF], sems[c % NBUF])
            cps[c].start()

        for c in range(NBUF - 1):
            start(c)
        for c in range(NCHUNK):
            if c + NBUF - 1 < NCHUNK:
                start(c + NBUF - 1)
            cps[c].wait()
            buf = bufs[c % NBUF]
            f = CHUNK_FIELD[c]
            w16 = ws_v[pl.ds(ebase + c * CHUNK, CHUNK)]
            wts = [_splat_lane(w16, t) for t in range(CHUNK)]

            overwrite = c in FIRST_CHUNK

            def ibody(i, _):
                sl = pl.ds(i * LANES, LANES)
                a = wts[0] * buf[0, sl]
                if not overwrite:
                    a = a + acc_v[f, sl]
                for t in range(1, CHUNK):
                    a = a + wts[t] * buf[t, sl]
                acc_v[f, sl] = a
                return 0

            lax.fori_loop(0, D // LANES, ibody, 0)
        pltpu.sync_copy(acc_v, out_hbm.at[base + e])
        return carry

    lax.fori_loop(0, EPW, ebody, 0)


GCH = 32


@functools.cache
def _get_assemble_kernel():
    return functools.partial(
        pl.kernel,
        mesh=plsc.VectorSubcoreMesh(core_axis_name="c", subcore_axis_name="s"),
        out_type=jax.ShapeDtypeStruct((BN, 133, D), _f32),
        scratch_types=[
            pltpu.VMEM((128 // GCH, GCH), jnp.int32),
            pltpu.VMEM((5, D), _f32),
            pltpu.VMEM((GCH, D), _f32),
            pltpu.VMEM((GCH, D), _f32),
            pltpu.SemaphoreType.DMA,
            pltpu.SemaphoreType.DMA,
            pltpu.SemaphoreType.DMA,
            pltpu.SemaphoreType.DMA,
        ],
        compiler_params=pltpu.CompilerParams(use_tc_tiling_on_sc=False),
    )(_assemble_body)


def _assemble_body(table2_hbm, didx_hbm, fc5_hbm, out_hbm,
                   didx_v, head_v, g0, g1, semh, semw, sem0, sem1):
    wid = lax.axis_index("s") * 2 + lax.axis_index("c")
    base = wid * EPW
    bufs = (g0, g1)
    sems = (sem0, sem1)
    nch = 128 // GCH

    def ebody(e, carry):
        b = base + e
        pltpu.sync_copy(didx_hbm.at[b], didx_v)
        cps = [None] * nch
        cps[0] = pltpu.make_async_copy(
            table2_hbm.at[didx_v.at[0]], bufs[0], sems[0])
        cps[0].start()
        cp_h = pltpu.make_async_copy(fc5_hbm.at[b], head_v, semh)
        cp_h.start()
        cps[1] = pltpu.make_async_copy(
            table2_hbm.at[didx_v.at[1]], bufs[1], sems[1])
        cps[1].start()
        cp_h.wait()
        cw_h = pltpu.make_async_copy(head_v, out_hbm.at[b, pl.ds(0, 5)], semw)
        cw_h.start()
        for c in range(nch):
            cps[c].wait()
            pltpu.sync_copy(bufs[c % 2], out_hbm.at[b, pl.ds(5 + c * GCH, GCH)])
            if c + 2 < nch:
                cps[c + 2] = pltpu.make_async_copy(
                    table2_hbm.at[didx_v.at[c + 2]],
                    bufs[c % 2], sems[c % 2])
                cps[c + 2].start()
        cw_h.wait()
        return carry

    lax.fori_loop(0, EPW, ebody, 0)


def kernel(field, price, rating, brand, name, category, description,
           emb_table, W_price, W_rating, W_fc, b_fc, W_lin):
    price = price.astype(_f32)
    rating = rating.astype(_f32)
    brand = brand.astype(jnp.int32)
    name = name.astype(jnp.int32)
    category = category.astype(jnp.int32)
    description = description.astype(jnp.int32)
    field = field.astype(jnp.int32)

    W_top = W_fc[:D]
    W_bot = W_fc[D:]

    brand_w = (brand != 1).astype(_f32)
    name_w = (name != 1).astype(_f32)
    cm = category != 1
    cm2 = jnp.any(cm, axis=-1)
    n2 = cm2.sum(axis=-1).astype(_f32)
    cm1 = jnp.any(cm2, axis=-1)
    n1 = cm1.sum(axis=-1).astype(_f32)
    cat_w = cm.astype(_f32) / (
        (n2[..., None, None] + 1e-6) * (n1[:, None, None, None] + 1e-6))
    zpad4 = jnp.zeros((BN, 4), _f32)
    w_all = jnp.concatenate(
        [brand_w, zpad4, name_w, cat_w.reshape(BN, 288)], axis=1)
    ipad4 = jnp.zeros((BN, 4), jnp.int32)
    idx_all = jnp.concatenate(
        [brand, ipad4, name, category.reshape(BN, 288)], axis=1)

    field_name = jnp.take(emb_table, field[:, 0], axis=0)
    fn8 = jnp.concatenate([field_name, jnp.zeros((2, D), _f32)], axis=0)

    names_fcb = pl.pallas_call(
        _names_fc_body,
        out_shape=jax.ShapeDtypeStruct((8, D), _f32),
    )(fn8, W_top, b_fc[None, :])

    tile_v = 512
    gv = pl.cdiv(VOCAB, tile_v)
    table2 = pl.pallas_call(
        _table2_body,
        grid=(gv,),
        in_specs=[
            pl.BlockSpec((1, D), lambda i: (0, 0)),
            pl.BlockSpec((tile_v, D), lambda i: (i, 0)),
            pl.BlockSpec((D, D), lambda i: (0, 0)),
            pl.BlockSpec((D, D), lambda i: (0, 0)),
        ],
        out_specs=pl.BlockSpec((tile_v, D), lambda i: (i, 0)),
        out_shape=jax.ShapeDtypeStruct((VOCAB, D), _f32),
    )(names_fcb[5:6], emb_table, W_bot, W_lin)

    pooled = _get_pooled_kernel()(
        emb_table, idx_all.reshape(-1), w_all.reshape(-1))

    bt = 256
    fc5 = pl.pallas_call(
        _fc5_body,
        grid=(BN // bt,),
        in_specs=[
            pl.BlockSpec((bt, 11), lambda i: (i, 0)),
            pl.BlockSpec((bt, 4), lambda i: (i, 0)),
            pl.BlockSpec((bt, 3, D), lambda i: (i, 0, 0)),
            pl.BlockSpec((8, D), lambda i: (0, 0)),
            pl.BlockSpec((11, D), lambda i: (0, 0)),
            pl.BlockSpec((4, D), lambda i: (0, 0)),
            pl.BlockSpec((D, D), lambda i: (0, 0)),
            pl.BlockSpec((D, D), lambda i: (0, 0)),
        ],
        out_specs=pl.BlockSpec((bt, 5, D), lambda i: (i, 0, 0)),
        out_shape=jax.ShapeDtypeStruct((BN, 5, D), _f32),
    )(price, rating, pooled, names_fcb, W_price, W_rating, W_bot, W_lin)

    didx = description.reshape(BN, 128 // GCH, GCH)
    all_embeddings = _get_assemble_kernel()(table2, didx, fc5)

    price_mask = price.sum(axis=1, keepdims=True) != 0.0
    ones = jnp.ones((BN, 1), bool)
    all_masks = jnp.concatenate(
        [price_mask, ones, brand[:, :1] != 1, name[:, :1] != 1, ones,
         description != 1], axis=1)

    return all_embeddings, all_masks
